# baseline (speedup 1.0000x reference)
import functools

import jax
import jax.numpy as jnp
from jax import lax
from jax.experimental import pallas as pl
from jax.experimental.pallas import tpu as pltpu
from jax.experimental.pallas import tpu_sc as plsc

_NUM_CORES = 2
_NUM_SUBCORES = 16
_NUM_WORKERS = _NUM_CORES * _NUM_SUBCORES

_CHUNK = 8
_NBUF = 8


@functools.partial(jax.jit, static_argnames=("b_per_w", "d"))
def _gather_rows(idx, table, *, b_per_w, d):
    b_total = idx.shape[0]
    nchunk = b_per_w // _CHUNK
    mesh = plsc.VectorSubcoreMesh(core_axis_name="c", subcore_axis_name="s")

    @functools.partial(
        pl.kernel,
        out_type=jax.ShapeDtypeStruct((b_total, d), jnp.float32),
        mesh=mesh,
        scratch_types=[
            pltpu.VMEM((b_per_w,), jnp.int32),
            pltpu.VMEM((_NBUF, _CHUNK, d), jnp.float32),
        ]
        + [pltpu.SemaphoreType.DMA] * (2 * _NBUF),
    )
    def body(idx_hbm, table_hbm, out_hbm, idx_v, rows_v, *sems):
        gsems = sems[:_NBUF]
        osems = sems[_NBUF:]
        wid = lax.axis_index("s") * _NUM_CORES + lax.axis_index("c")
        base = wid * b_per_w
        pltpu.sync_copy(idx_hbm.at[pl.ds(base, b_per_w)], idx_v)

        gd = [None] * nchunk
        od = [None] * nchunk

        def start_gather(c):
            buf = c % _NBUF
            gd[c] = pltpu.async_copy(
                table_hbm.at[idx_v.at[pl.ds(c * _CHUNK, _CHUNK)]],
                rows_v.at[buf],
                gsems[buf],
            )

        for c in range(min(_NBUF - 1, nchunk)):
            start_gather(c)
        for c in range(nchunk):
            nxt = c + _NBUF - 1
            if nxt < nchunk:
                if c >= 1:
                    od[c - 1].wait()
                start_gather(nxt)
            gd[c].wait()
            buf = c % _NBUF
            od[c] = pltpu.async_copy(
                rows_v.at[buf],
                out_hbm.at[pl.ds(base + c * _CHUNK, _CHUNK)],
                osems[buf],
            )
        for c in range(max(0, nchunk - _NBUF), nchunk):
            if od[c] is not None:
                od[c].wait()

    return body(idx, table)


def kernel(token_ids, table):
    b_total = token_ids.size
    d = table.shape[1]
    b_per_w = b_total // _NUM_WORKERS
    idx = token_ids.reshape(-1).astype(jnp.int32)
    out = _gather_rows(idx, table, b_per_w=b_per_w, d=d)
    return out.reshape(*token_ids.shape, d)


# speedup vs baseline: 1.0230x; 1.0230x over previous
import functools

import jax
import jax.numpy as jnp
from jax import lax
from jax.experimental import pallas as pl
from jax.experimental.pallas import tpu as pltpu
from jax.experimental.pallas import tpu_sc as plsc

_NUM_CORES = 2
_NUM_SUBCORES = 16
_NUM_WORKERS = _NUM_CORES * _NUM_SUBCORES

_CHUNK = 16
_NBUF = 4


@functools.partial(jax.jit, static_argnames=("b_per_w", "d"))
def _gather_rows(idx, table, *, b_per_w, d):
    b_total = idx.shape[0]
    nchunk = b_per_w // _CHUNK
    mesh = plsc.VectorSubcoreMesh(core_axis_name="c", subcore_axis_name="s")

    @functools.partial(
        pl.kernel,
        out_type=jax.ShapeDtypeStruct((b_total, d), jnp.float32),
        mesh=mesh,
        scratch_types=[
            pltpu.VMEM((b_per_w,), jnp.int32),
            pltpu.VMEM((_NBUF, _CHUNK, d), jnp.float32),
        ]
        + [pltpu.SemaphoreType.DMA] * (2 * _NBUF),
    )
    def body(idx_hbm, table_hbm, out_hbm, idx_v, rows_v, *sems):
        gsems = sems[:_NBUF]
        osems = sems[_NBUF:]
        wid = lax.axis_index("s") * _NUM_CORES + lax.axis_index("c")
        base = wid * b_per_w
        pltpu.sync_copy(idx_hbm.at[pl.ds(base, b_per_w)], idx_v)

        gd = [None] * nchunk
        od = [None] * nchunk

        def start_gather(c):
            buf = c % _NBUF
            gd[c] = pltpu.async_copy(
                table_hbm.at[idx_v.at[pl.ds(c * _CHUNK, _CHUNK)]],
                rows_v.at[buf],
                gsems[buf],
            )

        for c in range(min(_NBUF - 1, nchunk)):
            start_gather(c)
        for c in range(nchunk):
            nxt = c + _NBUF - 1
            if nxt < nchunk:
                if c >= 1:
                    od[c - 1].wait()
                start_gather(nxt)
            gd[c].wait()
            buf = c % _NBUF
            od[c] = pltpu.async_copy(
                rows_v.at[buf],
                out_hbm.at[pl.ds(base + c * _CHUNK, _CHUNK)],
                osems[buf],
            )
        for c in range(max(0, nchunk - _NBUF), nchunk):
            if od[c] is not None:
                od[c].wait()

    return body(idx, table)


def kernel(token_ids, table):
    b_total = token_ids.size
    d = table.shape[1]
    b_per_w = b_total // _NUM_WORKERS
    idx = token_ids.reshape(-1).astype(jnp.int32)
    out = _gather_rows(idx, table, b_per_w=b_per_w, d=d)
    return out.reshape(*token_ids.shape, d)


# speedup vs baseline: 1.0338x; 1.0106x over previous
import functools

import jax
import jax.numpy as jnp
from jax import lax
from jax.experimental import pallas as pl
from jax.experimental.pallas import tpu as pltpu
from jax.experimental.pallas import tpu_sc as plsc

_NUM_CORES = 2
_NUM_SUBCORES = 16
_NUM_WORKERS = _NUM_CORES * _NUM_SUBCORES

_CHUNK = 16
_NBUF = 5


@functools.partial(jax.jit, static_argnames=("b_per_w", "d"))
def _gather_rows(idx, table, *, b_per_w, d):
    b_total = idx.shape[0]
    nchunk = b_per_w // _CHUNK
    mesh = plsc.VectorSubcoreMesh(core_axis_name="c", subcore_axis_name="s")

    @functools.partial(
        pl.kernel,
        out_type=jax.ShapeDtypeStruct((b_total, d), jnp.float32),
        mesh=mesh,
        scratch_types=[
            pltpu.VMEM((b_per_w,), jnp.int32),
            pltpu.VMEM((_NBUF, _CHUNK, d), jnp.float32),
        ]
        + [pltpu.SemaphoreType.DMA] * (2 * _NBUF),
    )
    def body(idx_hbm, table_hbm, out_hbm, idx_v, rows_v, *sems):
        gsems = sems[:_NBUF]
        osems = sems[_NBUF:]
        wid = lax.axis_index("s") * _NUM_CORES + lax.axis_index("c")
        base = wid * b_per_w
        pltpu.sync_copy(idx_hbm.at[pl.ds(base, b_per_w)], idx_v)

        gd = [None] * nchunk
        od = [None] * nchunk

        def start_gather(c):
            buf = c % _NBUF
            gd[c] = pltpu.async_copy(
                table_hbm.at[idx_v.at[pl.ds(c * _CHUNK, _CHUNK)]],
                rows_v.at[buf],
                gsems[buf],
            )

        for c in range(min(_NBUF - 1, nchunk)):
            start_gather(c)
        for c in range(nchunk):
            nxt = c + _NBUF - 1
            if nxt < nchunk:
                if c >= 1:
                    od[c - 1].wait()
                start_gather(nxt)
            gd[c].wait()
            buf = c % _NBUF
            od[c] = pltpu.async_copy(
                rows_v.at[buf],
                out_hbm.at[pl.ds(base + c * _CHUNK, _CHUNK)],
                osems[buf],
            )
        for c in range(max(0, nchunk - _NBUF), nchunk):
            if od[c] is not None:
                od[c].wait()

    return body(idx, table)


def kernel(token_ids, table):
    b_total = token_ids.size
    d = table.shape[1]
    b_per_w = b_total // _NUM_WORKERS
    idx = token_ids.reshape(-1).astype(jnp.int32)
    out = _gather_rows(idx, table, b_per_w=b_per_w, d=d)
    return out.reshape(*token_ids.shape, d)
